# baseline (speedup 1.0000x reference)
import functools

import jax
import jax.numpy as jnp
from jax import lax
from jax.experimental import pallas as pl
from jax.experimental.pallas import tpu as pltpu
from jax.experimental.pallas import tpu_sc as plsc

_NT = 50000
_B = 1024
_BIT = 32
_NC = 10
_TW = 128
_ETA = 0.001
_CB = 5000
_NJ = _NT // _CB

_LOG2E = 1.4426950408889634
_LN2 = 0.6931471805599453
_DN = (((1,), (1,)), ((), ()))

_NW = 32
_BPW = _B // _NW


def _sc_gather(table, ind):
    mesh = plsc.VectorSubcoreMesh(core_axis_name="c", subcore_axis_name="s")

    @functools.partial(
        pl.kernel,
        mesh=mesh,
        out_type=jax.ShapeDtypeStruct((_B, _TW), jnp.float32),
        scratch_types=[
            pltpu.VMEM((_BPW,), jnp.int32),
            pltpu.VMEM((_BPW, _TW), jnp.float32),
            pltpu.SemaphoreType.DMA,
        ],
    )
    def gather_k(t_hbm, idx_hbm, g_hbm, idx_v, rows_v, sem):
        wid = lax.axis_index("s") * 2 + lax.axis_index("c")
        base = wid * _BPW
        pltpu.sync_copy(idx_hbm.at[pl.ds(base, _BPW)], idx_v)
        pltpu.async_copy(t_hbm.at[idx_v], rows_v, sem).wait()
        pltpu.sync_copy(rows_v, g_hbm.at[pl.ds(base, _BPW)])

    return gather_k(table, ind)


def _colsums(ip, sd):
    a = jnp.abs(ip)
    lg = jnp.log2(1.0 + jnp.exp2(a * (-_LOG2E)))
    cs_g = (jnp.sum(lg, axis=0, keepdims=True) * _LN2
            + 0.5 * jnp.sum(a, axis=0, keepdims=True))
    cs_sip = jnp.sum(jnp.where(sd > 0, ip, 0.0), axis=0, keepdims=True)
    return cs_g, cs_sip


def _main_kernel(u_ref, y_ref, U_ref, Y_ref, out_ref, tab_ref):
    j = pl.program_id(0)
    uh = u_ref[...] * 0.5
    uh16 = uh.astype(jnp.bfloat16)
    y16 = y_ref[...].astype(jnp.bfloat16)
    ush16 = jnp.sum(uh, axis=0, keepdims=True).astype(jnp.bfloat16)
    Ub16 = U_ref[...]
    Yb16 = Y_ref[...]
    tab_ref[:, 0:_BIT] = Ub16.astype(jnp.float32)
    tab_ref[:, _BIT:_BIT + _NC] = Yb16.astype(jnp.float32)
    ip = jax.lax.dot_general(uh16, Ub16, _DN,
                             preferred_element_type=jnp.float32)
    sd = jax.lax.dot_general(y16, Yb16, _DN,
                             preferred_element_type=jnp.float32)
    cs_g, cs_sip = _colsums(ip, sd)
    cs_ip = jax.lax.dot_general(ush16, Ub16, _DN,
                                preferred_element_type=jnp.float32)
    contrib = jnp.sum(cs_g + 0.5 * cs_ip - cs_sip)

    @pl.when(j == 0)
    def _first():
        out_ref[...] = jnp.full((1, 1), contrib, jnp.float32)

    @pl.when(j != 0)
    def _rest():
        out_ref[...] = out_ref[...] + contrib


def _corr_kernel(u_ref, y_ref, indc_ref, indr_ref, g_ref,
                 acc_ref, out_ref):
    u = u_ref[...]
    uh = u * 0.5
    uh16 = uh.astype(jnp.bfloat16)
    y16 = y_ref[...].astype(jnp.bfloat16)
    ush = jnp.sum(uh, axis=0, keepdims=True)
    ind_c = indc_ref[...]
    ind_r = indr_ref[...]
    ii = jax.lax.broadcasted_iota(jnp.int32, (_B, _B), 0)
    jj = jax.lax.broadcasted_iota(jnp.int32, (_B, _B), 1)
    winner = jnp.min(
        jnp.where((ind_c == ind_r) & (ii > jj), 0.0, 1.0),
        axis=0, keepdims=True)

    g = g_ref[...]
    Z = jnp.concatenate([g[:, :_BIT], u], axis=0)
    Zy = jnp.concatenate([g[:, _BIT:_BIT + _NC], y_ref[...]], axis=0)
    w2 = jnp.concatenate([-winner, winner], axis=1)
    ip_z = jax.lax.dot_general(uh16, Z.astype(jnp.bfloat16), _DN,
                               preferred_element_type=jnp.float32)
    sd_z = jax.lax.dot_general(y16, Zy.astype(jnp.bfloat16), _DN,
                               preferred_element_type=jnp.float32)
    cs_gz, cs_sipz = _colsums(ip_z, sd_z)
    cs_ipz = jax.lax.dot_general(ush, Z, _DN,
                                 preferred_element_type=jnp.float32)
    corr = jnp.sum((cs_gz + 0.5 * cs_ipz - cs_sipz) * w2)

    quant = jnp.sum((u - jnp.sign(u)) ** 2) * (_ETA * _NT / _BIT)
    out_ref[...] = (acc_ref[...] + (corr + quant)) * (
        1.0 / (_B * _NT))


def kernel(u, y, ind, U, Y):
    ind = ind.astype(jnp.int32)
    ind_c = ind.reshape(_B, 1)
    ind_r = ind.reshape(1, _B)

    acc, table = pl.pallas_call(
        _main_kernel,
        grid=(_NJ,),
        in_specs=[
            pl.BlockSpec((_B, _BIT), lambda j: (0, 0)),
            pl.BlockSpec((_B, _NC), lambda j: (0, 0)),
            pl.BlockSpec((_CB, _BIT), lambda j: (j, 0)),
            pl.BlockSpec((_CB, _NC), lambda j: (j, 0)),
        ],
        out_specs=(
            pl.BlockSpec((1, 1), lambda j: (0, 0)),
            pl.BlockSpec((_CB, _TW), lambda j: (j, 0)),
        ),
        out_shape=(
            jax.ShapeDtypeStruct((1, 1), jnp.float32),
            jax.ShapeDtypeStruct((_NT, _TW), jnp.float32),
        ),
    )(u, y, U.astype(jnp.bfloat16), Y.astype(jnp.bfloat16))
    g = _sc_gather(table, ind)

    total = pl.pallas_call(
        _corr_kernel,
        out_shape=jax.ShapeDtypeStruct((1, 1), jnp.float32),
    )(u, y, ind_c, ind_r, g, acc)
    return total[0, 0]
